# baseline (speedup 1.0000x reference)
import functools

import jax
import jax.numpy as jnp
from jax import lax
from jax.experimental import pallas as pl
from jax.experimental.pallas import tpu as pltpu
from jax.experimental.pallas import tpu_sc as plsc

_EMBED = [(33, 17), (2, 1), (8, 4), (3, 2), (3, 2), (3, 2), (3, 2), (3, 2), (20, 10)]
_FOFF = [0, 561, 563, 595, 601, 607, 613, 619, 625]
_B = 16384
_BLK = 2048
_NE = 48
_NW = 32
_BPW = _B // _NW
_CHUNK = 16
_NCHUNK = _BPW // _CHUNK

_COL_TAB = []
for _i, (_v, _d) in enumerate(_EMBED):
    for _c in range(_d):
        _COL_TAB.append((_FOFF[_i] + _c, _d, _i))


def _sc_body(t1d_hbm, i0, i1, i2, i3, i4, i5, i6, i7, i8, out_hbm,
             t1d_v, iv0, iv1, iv2, iv3, iv4, iv5, iv6, iv7, iv8, e_v):
    idx_hbm = [i0, i1, i2, i3, i4, i5, i6, i7, i8]
    iv = [iv0, iv1, iv2, iv3, iv4, iv5, iv6, iv7, iv8]
    wid = lax.axis_index("s") * 2 + lax.axis_index("c")
    base = wid * _BPW
    pltpu.sync_copy(t1d_hbm, t1d_v)
    for i in range(9):
        pltpu.sync_copy(idx_hbm[i].at[pl.ds(base, _BPW)], iv[i])

    lane = lax.broadcasted_iota(jnp.int32, (16,), 0)

    def chunk(k, carry):
        rowid = lane + k * _CHUNK
        idxv = [iv[i][pl.ds(k * _CHUNK, _CHUNK)] for i in range(9)]
        for j, (fb, d, i) in enumerate(_COL_TAB):
            addr = idxv[i] * d + fb
            vals = plsc.load_gather(t1d_v, [addr])
            jcol = jnp.full((16,), j, jnp.int32)
            plsc.store_scatter(e_v, [rowid, jcol], vals)
        return carry

    lax.fori_loop(0, _NCHUNK, chunk, 0)
    pltpu.sync_copy(e_v, out_hbm.at[pl.ds(base, _BPW)])


def _sc_gather(t1d, idxs):
    mesh = plsc.VectorSubcoreMesh(core_axis_name="c", subcore_axis_name="s")
    f = pl.kernel(
        _sc_body, mesh=mesh,
        compiler_params=pltpu.CompilerParams(needs_layout_passes=False),
        out_type=jax.ShapeDtypeStruct((_B, _NE), jnp.float32),
        scratch_types=(
            [pltpu.VMEM((1024,), jnp.float32)]
            + [pltpu.VMEM((_BPW,), jnp.int32) for _ in range(9)]
            + [pltpu.VMEM((_BPW, _NE), jnp.float32)]
        ),
    )
    return f(t1d, *idxs)


def _tc_body(e_ref, cont_ref, cont1_ref, wt_ref, wc_ref, bias_ref,
             gamma_ref, beta_ref, out_ref):
    c = cont_ref[...]
    mean = jnp.mean(c)
    var = jnp.mean(c * c) - mean * mean
    inv = jax.lax.rsqrt(var + 1e-5)
    cn = (cont1_ref[...] - mean) * inv * gamma_ref[...] + beta_ref[...]

    e = e_ref[...].astype(jnp.bfloat16)
    z = jnp.dot(e, wt_ref[...], preferred_element_type=jnp.float32)
    z = z + jax.lax.dot_general(cn, wc_ref[...],
                                dimension_numbers=(((0,), (0,)), ((), ())),
                                preferred_element_type=jnp.float32)
    z = z + bias_ref[...]
    mx = jnp.max(z, axis=1, keepdims=True)
    ex = jnp.exp(z - mx)
    out_ref[...] = ex / jnp.sum(ex, axis=1, keepdims=True)


def kernel(x, emb0, emb1, emb2, emb3, emb4, emb5, emb6, emb7, emb8, W, b,
           gamma, beta):
    tables = [emb0, emb1, emb2, emb3, emb4, emb5, emb6, emb7, emb8]
    t1d = jnp.concatenate([t.reshape(-1) for t in tables]
                          + [jnp.zeros((1024 - 825,), jnp.float32)])
    idxs = [x[:, 1 + i].astype(jnp.int32) for i in range(9)]

    e = _sc_gather(t1d, idxs)

    wt = jnp.zeros((_NE, 256), jnp.float32).at[:42, :].set(W[:, :42].T)
    cont_full = x[:, 0].reshape(8, 2048)
    cont1 = x[:, 0].reshape(1, _B)
    grid = _B // _BLK

    out = pl.pallas_call(
        _tc_body,
        grid=(grid,),
        in_specs=[
            pl.BlockSpec((_BLK, _NE), lambda j: (j, 0)),
            pl.BlockSpec((8, 2048), lambda j: (0, 0)),
            pl.BlockSpec((1, _BLK), lambda j: (0, j)),
            pl.BlockSpec((_NE, 256), lambda j: (0, 0)),
            pl.BlockSpec((1, 256), lambda j: (0, 0)),
            pl.BlockSpec((1, 256), lambda j: (0, 0)),
            pl.BlockSpec((1, 1), lambda j: (0, 0)),
            pl.BlockSpec((1, 1), lambda j: (0, 0)),
        ],
        out_specs=pl.BlockSpec((_BLK, 256), lambda j: (j, 0)),
        out_shape=jax.ShapeDtypeStruct((_B, 256), jnp.float32),
    )(e, cont_full, cont1, wt.astype(jnp.bfloat16), W[:, 42].reshape(1, 256),
      b.reshape(1, 256), gamma.reshape(1, 1), beta.reshape(1, 1))
    return out
